# baseline (speedup 1.0000x reference)
import jax
import jax.numpy as jnp
from jax import lax
from jax.experimental import pallas as pl
from jax.experimental.pallas import tpu as pltpu
from jax.experimental.pallas import tpu_sc as plsc

NA = 64
ROW_F = 2 * NA
B = 16384
L = 16
NW = 32
PER_W = B // NW
BLK = 128
NBLK = B // BLK
CHUNK = BLK
N_CHUNKS = PER_W // CHUNK


_GATHER_DNUMS = lax.GatherDimensionNumbers(
    offset_dims=(), collapsed_slice_dims=(0,), start_index_map=(0,))


def _permute16(x, idx):
  return lax.gather(
      x, idx[:, None], _GATHER_DNUMS, slice_sizes=(1,),
      mode=lax.GatherScatterMode.PROMISE_IN_BOUNDS)


def _rsqrt16(x):
  xi = lax.bitcast_convert_type(x, jnp.int32)
  yi = jnp.int32(0x5F3759DF) - (xi >> 1)
  y = lax.bitcast_convert_type(yi, jnp.float32)
  xh = x * jnp.float32(0.5)
  for _ in range(2):
    y = y * (jnp.float32(1.5) - xh * y * y)
  return y


def _sc_body(idx_hbm, cb_hbm, out_hbm,
             idx_v, rows0, rows1, outb0, outb1,
             sem_g0, sem_g1, sem_o0, sem_o1):
  nc = 2
  wid = lax.axis_index("s") * nc + lax.axis_index("c")
  base = wid * PER_W
  iota = lax.iota(jnp.int32, L)

  rows = (rows0, rows1)
  outb = (outb0, outb1)
  sem_g = (sem_g0, sem_g1)
  sem_o = (sem_o0, sem_o1)

  pltpu.sync_copy(idx_hbm.at[pl.ds(base, PER_W)], idx_v)

  def fire_gather(c):
    p = c % 2
    pltpu.make_async_copy(
        cb_hbm.at[idx_v.at[pl.ds(c * CHUNK, CHUNK)]], rows[p],
        sem_g[p]).start()

  def drain_gather(c):
    p = c % 2
    pltpu.make_async_copy(cb_hbm.at[pl.ds(0, CHUNK), :], rows[p],
                          sem_g[p]).wait()

  def compute(c):
    p = c % 2

    def row_body(j, carry):
      rs = [rows[p][j, pl.ds(L * k, L)] for k in range(NA // L)]
      im = [rows[p][j, pl.ds(NA + L * k, L)] for k in range(NA // L)]
      ss = rs[0] * rs[0]
      for v in rs[1:]:
        ss = ss + v * v
      for v in im:
        ss = ss + v * v
      for sh in (8, 4, 2, 1):
        ss = ss + _permute16(ss, iota ^ sh)
      scale = _rsqrt16(ss)
      for k in range(NA // L):
        col = j + (iota + (L * k)) * (2 * BLK)
        plsc.store_scatter(outb[p], [col], rs[k] * scale)
        plsc.store_scatter(outb[p], [col + BLK], im[k] * scale)
      return carry

    lax.fori_loop(0, CHUNK, row_body, 0, unroll=4)

  def fire_out(c):
    p = c % 2
    tglob = wid * N_CHUNKS + c
    for a in range(NA):
      pltpu.make_async_copy(
          outb[p].at[pl.ds(a * (2 * BLK), 2 * BLK)],
          out_hbm.at[pl.ds(a * (2 * BLK * NBLK) + tglob * (2 * BLK),
                           2 * BLK)],
          sem_o[p]).start()

  def drain_out(c):
    p = c % 2
    pltpu.make_async_copy(out_hbm.at[pl.ds(0, ROW_F * BLK)], outb[p],
                          sem_o[p]).wait()

  fire_gather(0)
  for c in range(N_CHUNKS):
    if c + 1 < N_CHUNKS:
      fire_gather(c + 1)
    drain_gather(c)
    if c >= 2:
      drain_out(c - 2)
    compute(c)
    fire_out(c)
  drain_out(N_CHUNKS - 2)
  drain_out(N_CHUNKS - 1)


@jax.jit
def _sc_call(beam_index, cb):
  mesh = plsc.VectorSubcoreMesh(core_axis_name="c", subcore_axis_name="s")
  f = pl.kernel(
      _sc_body,
      out_type=jax.ShapeDtypeStruct((B * ROW_F,), jnp.float32),
      mesh=mesh,
      compiler_params=pltpu.CompilerParams(
          needs_layout_passes=False, use_tc_tiling_on_sc=True),
      scratch_types=[
          pltpu.VMEM((PER_W,), jnp.int32),
          pltpu.VMEM((CHUNK, ROW_F), jnp.float32),
          pltpu.VMEM((CHUNK, ROW_F), jnp.float32),
          pltpu.VMEM((ROW_F * BLK,), jnp.float32),
          pltpu.VMEM((ROW_F * BLK,), jnp.float32),
          pltpu.SemaphoreType.DMA,
          pltpu.SemaphoreType.DMA,
          pltpu.SemaphoreType.DMA,
          pltpu.SemaphoreType.DMA,
      ],
  )
  return f(beam_index, cb)


def kernel(beam_index, codebook_real, codebook_imag):
  cb = jnp.concatenate([codebook_real, codebook_imag], axis=1)
  out = _sc_call(beam_index, cb)
  out = out.reshape(NA, NBLK, 2, BLK)
  out = out.transpose(1, 3, 0, 2)
  return out.reshape(B, NA, 2)
